# baseline (speedup 1.0000x reference)
import functools

import jax
import jax.numpy as jnp
from jax import lax
from jax.experimental import pallas as pl
from jax.experimental.pallas import tpu as pltpu
from jax.experimental.pallas import tpu_sc as plsc

VOCAB = 1000000
EMBED_DIM = 64
BATCH = 4096
HIST = 200

_TOTAL = BATCH * HIST
_NW = 32
_PER_W = _TOTAL // _NW
_CHUNK = 128
_NCHUNK = _PER_W // _CHUNK

_mesh = plsc.VectorSubcoreMesh(core_axis_name="c", subcore_axis_name="s")


@functools.partial(
    pl.kernel,
    mesh=_mesh,
    out_type=jax.ShapeDtypeStruct((_TOTAL, EMBED_DIM), jnp.float32),
    scratch_types=[
        pltpu.VMEM((_CHUNK,), jnp.int32),
        pltpu.VMEM((_CHUNK, EMBED_DIM), jnp.float32),
        pltpu.SemaphoreType.DMA,
    ],
    compiler_params=pltpu.CompilerParams(use_tc_tiling_on_sc=False),
)
def _gather_kernel(idx_hbm, table_hbm, out_hbm, idx_v, rows_v, sem):
    wid = lax.axis_index("s") * 2 + lax.axis_index("c")
    base = wid * _PER_W

    def body(i, _):
        off = base + i * _CHUNK
        pltpu.sync_copy(idx_hbm.at[pl.ds(off, _CHUNK)], idx_v)
        pltpu.async_copy(table_hbm.at[idx_v], rows_v, sem).wait()
        pltpu.sync_copy(rows_v, out_hbm.at[pl.ds(off, _CHUNK)])
        return 0

    lax.fori_loop(0, _NCHUNK, body, 0)


def kernel(idx_texts, table):
    flat_idx = idx_texts.reshape(_TOTAL).astype(jnp.int32)
    out = _gather_kernel(flat_idx, table)
    return out.reshape(BATCH, HIST, EMBED_DIM)


# speedup vs baseline: 1.1750x; 1.1750x over previous
import functools

import jax
import jax.numpy as jnp
from jax import lax
from jax.experimental import pallas as pl
from jax.experimental.pallas import tpu as pltpu
from jax.experimental.pallas import tpu_sc as plsc

VOCAB = 1000000
EMBED_DIM = 64
BATCH = 4096
HIST = 200

_TOTAL = BATCH * HIST
_NW = 32
_PER_W = _TOTAL // _NW
_CHUNK = 1024
_NCHUNK = _PER_W // _CHUNK

_mesh = plsc.VectorSubcoreMesh(core_axis_name="c", subcore_axis_name="s")


@functools.partial(
    pl.kernel,
    mesh=_mesh,
    out_type=jax.ShapeDtypeStruct((_TOTAL, EMBED_DIM), jnp.float32),
    scratch_types=[
        pltpu.VMEM((_CHUNK,), jnp.int32),
        pltpu.VMEM((_CHUNK, EMBED_DIM), jnp.float32),
        pltpu.SemaphoreType.DMA,
    ],
    compiler_params=pltpu.CompilerParams(use_tc_tiling_on_sc=False),
)
def _gather_kernel(idx_hbm, table_hbm, out_hbm, idx_v, rows_v, sem):
    wid = lax.axis_index("s") * 2 + lax.axis_index("c")
    base = wid * _PER_W

    def body(i, _):
        off = base + i * _CHUNK
        pltpu.sync_copy(idx_hbm.at[pl.ds(off, _CHUNK)], idx_v)
        pltpu.async_copy(table_hbm.at[idx_v], rows_v, sem).wait()
        pltpu.sync_copy(rows_v, out_hbm.at[pl.ds(off, _CHUNK)])
        return 0

    lax.fori_loop(0, _NCHUNK, body, 0)


def kernel(idx_texts, table):
    flat_idx = idx_texts.reshape(_TOTAL).astype(jnp.int32)
    out = _gather_kernel(flat_idx, table)
    return out.reshape(BATCH, HIST, EMBED_DIM)


# speedup vs baseline: 1.1943x; 1.0164x over previous
import functools

import jax
import jax.numpy as jnp
from jax import lax
from jax.experimental import pallas as pl
from jax.experimental.pallas import tpu as pltpu
from jax.experimental.pallas import tpu_sc as plsc

VOCAB = 1000000
EMBED_DIM = 64
BATCH = 4096
HIST = 200

_TOTAL = BATCH * HIST
_NW = 32
_PER_W = _TOTAL // _NW
_CHUNK = 512
_NCHUNK = _PER_W // _CHUNK

_mesh = plsc.VectorSubcoreMesh(core_axis_name="c", subcore_axis_name="s")


@functools.partial(
    pl.kernel,
    mesh=_mesh,
    out_type=jax.ShapeDtypeStruct((_TOTAL, EMBED_DIM), jnp.float32),
    scratch_types=[
        pltpu.VMEM((_PER_W,), jnp.int32),
        pltpu.VMEM((_CHUNK, EMBED_DIM), jnp.float32),
        pltpu.VMEM((_CHUNK, EMBED_DIM), jnp.float32),
        pltpu.SemaphoreType.DMA,
        pltpu.SemaphoreType.DMA,
    ],
    compiler_params=pltpu.CompilerParams(use_tc_tiling_on_sc=False),
)
def _gather_kernel(idx_hbm, table_hbm, out_hbm, idx_v, rows0, rows1, sem0, sem1):
    wid = lax.axis_index("s") * 2 + lax.axis_index("c")
    base = wid * _PER_W

    pltpu.sync_copy(idx_hbm.at[pl.ds(base, _PER_W)], idx_v)

    def gather(g, rows, sem):
        return pltpu.make_async_copy(
            table_hbm.at[idx_v.at[pl.ds(g * _CHUNK, _CHUNK)]], rows, sem)

    def write(g, rows):
        pltpu.sync_copy(rows, out_hbm.at[pl.ds(base + g * _CHUNK, _CHUNK)])

    gather(0, rows0, sem0).start()

    def body(i, _):
        g = 2 * i
        gather(g, rows0, sem0).wait()
        gather(g + 1, rows1, sem1).start()
        write(g, rows0)
        gather(g + 1, rows1, sem1).wait()

        @pl.when(g + 2 < _NCHUNK)
        def _():
            gather(g + 2, rows0, sem0).start()

        write(g + 1, rows1)
        return 0

    lax.fori_loop(0, _NCHUNK // 2, body, 0)


def kernel(idx_texts, table):
    flat_idx = idx_texts.reshape(_TOTAL).astype(jnp.int32)
    out = _gather_kernel(flat_idx, table)
    return out.reshape(BATCH, HIST, EMBED_DIM)


# speedup vs baseline: 1.1992x; 1.0041x over previous
import functools

import jax
import jax.numpy as jnp
from jax import lax
from jax.experimental import pallas as pl
from jax.experimental.pallas import tpu as pltpu
from jax.experimental.pallas import tpu_sc as plsc

VOCAB = 1000000
EMBED_DIM = 64
BATCH = 4096
HIST = 200

_TOTAL = BATCH * HIST
_NW = 32
_PER_W = _TOTAL // _NW
_CHUNK = 256
_NCHUNK = _PER_W // _CHUNK
_NBUF = 4
_DEPTH = 3

_mesh = plsc.VectorSubcoreMesh(core_axis_name="c", subcore_axis_name="s")


@functools.partial(
    pl.kernel,
    mesh=_mesh,
    out_type=jax.ShapeDtypeStruct((_TOTAL, EMBED_DIM), jnp.float32),
    scratch_types=[
        pltpu.VMEM((_PER_W,), jnp.int32),
        [pltpu.VMEM((_CHUNK, EMBED_DIM), jnp.float32) for _ in range(_NBUF)],
        [pltpu.SemaphoreType.DMA for _ in range(_NBUF)],
        [pltpu.SemaphoreType.DMA for _ in range(_NBUF)],
    ],
    compiler_params=pltpu.CompilerParams(use_tc_tiling_on_sc=False),
)
def _gather_kernel(idx_hbm, table_hbm, out_hbm, idx_v, rows, gsem, wsem):
    wid = lax.axis_index("s") * 2 + lax.axis_index("c")
    base = wid * _PER_W

    pltpu.sync_copy(idx_hbm.at[pl.ds(base, _PER_W)], idx_v)

    def gather(g, b):
        return pltpu.make_async_copy(
            table_hbm.at[idx_v.at[pl.ds(g * _CHUNK, _CHUNK)]], rows[b], gsem[b])

    def write(g, b):
        return pltpu.make_async_copy(
            rows[b], out_hbm.at[pl.ds(base + g * _CHUNK, _CHUNK)], wsem[b])

    for g in range(_DEPTH):
        gather(g, g % _NBUF).start()

    def body(i, _):
        for k in range(_NBUF):
            g = i * _NBUF + k
            b_w = k
            b_g = (k + _DEPTH) % _NBUF
            gather(g, b_w).wait()
            write(g, b_w).start()

            @pl.when(g + _DEPTH < _NCHUNK)
            def _():
                @pl.when(g + _DEPTH >= _NBUF)
                def _():
                    write(g, b_g).wait()
                gather(g + _DEPTH, b_g).start()
        return 0

    lax.fori_loop(0, _NCHUNK // _NBUF, body, 0)

    for b in range(_NBUF):
        write(0, b).wait()


def kernel(idx_texts, table):
    flat_idx = idx_texts.reshape(_TOTAL).astype(jnp.int32)
    out = _gather_kernel(flat_idx, table)
    return out.reshape(BATCH, HIST, EMBED_DIM)


# speedup vs baseline: 1.7741x; 1.4794x over previous
import functools

import jax
import jax.numpy as jnp
from jax import lax
from jax.experimental import pallas as pl
from jax.experimental.pallas import tpu as pltpu
from jax.experimental.pallas import tpu_sc as plsc

VOCAB = 1000000
EMBED_DIM = 64
BATCH = 4096
HIST = 200

_TOTAL = BATCH * HIST
_NW = 32
_PER_W = _TOTAL // _NW
_CHUNK = 256
_NCHUNK = _PER_W // _CHUNK

_mesh = plsc.VectorSubcoreMesh(core_axis_name="c", subcore_axis_name="s")


@functools.partial(
    pl.kernel,
    mesh=_mesh,
    out_type=jax.ShapeDtypeStruct((_TOTAL, EMBED_DIM), jnp.float32),
    scratch_types=[
        pltpu.VMEM((_PER_W,), jnp.int32),
        [pltpu.VMEM((_CHUNK, EMBED_DIM), jnp.float32) for _ in range(2)],
        [pltpu.SemaphoreType.DMA for _ in range(2)],
        [pltpu.SemaphoreType.DMA for _ in range(2)],
    ],
    compiler_params=pltpu.CompilerParams(use_tc_tiling_on_sc=True),
)
def _gather_kernel(idx_hbm, table_hbm, out_hbm, idx_v, rows, gsem, wsem):
    wid = lax.axis_index("s") * 2 + lax.axis_index("c")
    base = wid * _PER_W

    pltpu.sync_copy(idx_hbm.at[pl.ds(base, _PER_W)], idx_v)

    def fire_chunk(g, b):
        def grp(j, _):
            v16 = idx_v[pl.ds(g * _CHUNK + j * 16, 16)]
            for t in range(16):
                pltpu.make_async_copy(
                    table_hbm.at[pl.ds(v16[t], 1)],
                    rows[b].at[pl.ds(j * 16 + t, 1)],
                    gsem[b],
                ).start()
            return 0

        lax.fori_loop(0, _CHUNK // 16, grp, 0)

    def drain_chunk(b):
        pltpu.make_async_copy(
            table_hbm.at[pl.ds(0, _CHUNK)], rows[b], gsem[b]
        ).wait()

    def write_chunk(g, b):
        return pltpu.make_async_copy(
            rows[b], out_hbm.at[pl.ds(base + g * _CHUNK, _CHUNK)], wsem[b]
        )

    fire_chunk(0, 0)

    def body(i, _):
        for k in range(2):
            g = 2 * i + k
            b = k
            drain_chunk(b)
            write_chunk(g, b).start()

            @pl.when(g + 1 < _NCHUNK)
            def _():
                @pl.when(g >= 1)
                def _():
                    write_chunk(g, 1 - b).wait()
                fire_chunk(g + 1, 1 - b)

        return 0

    lax.fori_loop(0, _NCHUNK // 2, body, 0)

    write_chunk(0, 0).wait()
    write_chunk(0, 1).wait()


def kernel(idx_texts, table):
    flat_idx = idx_texts.reshape(_TOTAL).astype(jnp.int32)
    out = _gather_kernel(flat_idx, table)
    return out.reshape(BATCH, HIST, EMBED_DIM)
